# baseline (speedup 1.0000x reference)
import functools

import jax
import jax.numpy as jnp
import ml_dtypes
import numpy as np
from jax import lax
from jax.experimental import pallas as pl
from jax.experimental.pallas import tpu as pltpu
from jax.experimental.pallas import tpu_sc as plsc

MAX_LEN = 4096
D_MODEL = 2048

_NC = 2
_NS = 16
_NW = _NC * _NS

_CHUNK = 8
_L = 16


def _pos_encoding(max_len, d_model):
    pos = np.arange(max_len, dtype=np.float32)[:, None]
    _2i = np.arange(0, d_model, 2, dtype=np.float32)
    angle = pos / np.power(np.float32(10000.0), _2i / np.float32(d_model))
    enc = np.zeros((max_len, d_model), dtype=np.float32)
    enc[:, 0::2] = np.sin(angle)
    enc[:, 1::2] = np.cos(angle)
    return enc


def _shuffle_for_widening(enc):
    S, D = enc.shape
    blk = enc.reshape(S, D // 32, 2, 16)
    shuf = blk.transpose(0, 1, 3, 2).reshape(S, D)
    return shuf.astype(ml_dtypes.bfloat16).view(np.int32)


_POS_ENC = _shuffle_for_widening(_pos_encoding(MAX_LEN, D_MODEL))


def kernel(x, table):
    B, S = x.shape
    V, D = table.shape
    N = B * S
    pos = jnp.asarray(_POS_ENC[:S])

    n_per_w = N // _NW
    w_per_b = S // n_per_w
    C = _CHUNK
    n_chunks = n_per_w // C

    mesh = plsc.VectorSubcoreMesh(core_axis_name="c", subcore_axis_name="s")

    @functools.partial(
        pl.kernel,
        out_type=jax.ShapeDtypeStruct((N, D), jnp.float32),
        mesh=mesh,
        scratch_types=[
            pltpu.VMEM((n_per_w,), jnp.int32),
            [pltpu.VMEM((C, D), jnp.float32)] * 3,
            [pltpu.VMEM((C, D // 2), jnp.int32)] * 3,
            [pltpu.SemaphoreType.DMA] * 3,
            [pltpu.SemaphoreType.DMA] * 3,
            [pltpu.SemaphoreType.DMA] * 3,
        ],
    )
    def emb_kernel(x_hbm, table_hbm, pos_hbm, out_hbm,
                   idx_v, bufs, pbufs, gsems, psems, osems):
        wid = lax.axis_index("s") * _NC + lax.axis_index("c")
        bi = wid // w_per_b
        s0 = lax.rem(wid, w_per_b) * n_per_w

        pltpu.sync_copy(x_hbm.at[bi, pl.ds(s0, n_per_w)], idx_v)

        def start_chunk(ci):
            p = ci % 3
            off = ci * C
            g = pltpu.async_copy(
                table_hbm.at[idx_v.at[pl.ds(off, C)]], bufs[p], gsems[p])
            pp = pltpu.async_copy(
                pos_hbm.at[pl.ds(s0 + off, C)], pbufs[p], psems[p])
            return g, pp

        def add_pos(p):
            buf, pbuf = bufs[p], pbufs[p]
            bpr = D // (2 * _L)

            @plsc.parallel_loop(0, bpr, 1, unroll=2)
            def _(i):
                col = i * 2 * _L
                for r in range(C):
                    w = pbuf[r, pl.ds(i * _L, _L)]
                    a = lax.bitcast_convert_type(
                        lax.shift_left(w, 16), jnp.float32)
                    b = lax.bitcast_convert_type(
                        lax.bitwise_and(w, jnp.int32(-65536)), jnp.float32)
                    plsc.addupdate(buf.at[r, pl.ds(col, _L)], a)
                    plsc.addupdate(buf.at[r, pl.ds(col + _L, _L)], b)

        store_descs = [None, None, None]
        in_flight = [start_chunk(0), start_chunk(1)]
        for ci in range(n_chunks):
            p = ci % 3
            if ci + 2 < n_chunks:
                nxt = (ci + 2) % 3
                if store_descs[nxt] is not None:
                    store_descs[nxt].wait()
                in_flight.append(start_chunk(ci + 2))
            g, pp = in_flight[ci]
            g.wait()
            pp.wait()
            add_pos(p)
            store_descs[p] = pltpu.async_copy(
                bufs[p], out_hbm.at[pl.ds(wid * n_per_w + ci * C, C)], osems[p])
        for d in store_descs:
            if d is not None:
                d.wait()

    return emb_kernel(x, table, pos).reshape(B, S, D)


# speedup vs baseline: 1.0103x; 1.0103x over previous
import functools

import jax
import jax.numpy as jnp
import ml_dtypes
import numpy as np
from jax import lax
from jax.experimental import pallas as pl
from jax.experimental.pallas import tpu as pltpu
from jax.experimental.pallas import tpu_sc as plsc

MAX_LEN = 4096
D_MODEL = 2048

_NC = 2
_NS = 16
_NW = _NC * _NS

_CHUNK = 8
_L = 16


def _pos_encoding(max_len, d_model):
    pos = np.arange(max_len, dtype=np.float32)[:, None]
    _2i = np.arange(0, d_model, 2, dtype=np.float32)
    angle = pos / np.power(np.float32(10000.0), _2i / np.float32(d_model))
    enc = np.zeros((max_len, d_model), dtype=np.float32)
    enc[:, 0::2] = np.sin(angle)
    enc[:, 1::2] = np.cos(angle)
    return enc


def _shuffle_for_widening(enc):
    S, D = enc.shape
    blk = enc.reshape(S, D // 32, 2, 16)
    shuf = blk.transpose(0, 1, 3, 2).reshape(S, D)
    return shuf.astype(ml_dtypes.bfloat16).view(np.int32)


_POS_ENC = _shuffle_for_widening(_pos_encoding(MAX_LEN, D_MODEL))


def kernel(x, table):
    B, S = x.shape
    V, D = table.shape
    N = B * S
    pos = jnp.asarray(_POS_ENC[:S])

    n_per_w = N // _NW
    w_per_b = S // n_per_w
    C = _CHUNK
    n_chunks = n_per_w // C

    mesh = plsc.VectorSubcoreMesh(core_axis_name="c", subcore_axis_name="s")

    @functools.partial(
        pl.kernel,
        out_type=jax.ShapeDtypeStruct((N, D), jnp.float32),
        mesh=mesh,
        scratch_types=[
            pltpu.VMEM((n_per_w,), jnp.int32),
            [pltpu.VMEM((C, D), jnp.float32)] * 4,
            [pltpu.VMEM((C, D // 2), jnp.int32)] * 4,
            [pltpu.SemaphoreType.DMA] * 4,
            [pltpu.SemaphoreType.DMA] * 4,
            [pltpu.SemaphoreType.DMA] * 4,
        ],
    )
    def emb_kernel(x_hbm, table_hbm, pos_hbm, out_hbm,
                   idx_v, bufs, pbufs, gsems, psems, osems):
        wid = lax.axis_index("s") * _NC + lax.axis_index("c")
        bi = wid // w_per_b
        s0 = lax.rem(wid, w_per_b) * n_per_w

        pltpu.sync_copy(x_hbm.at[bi, pl.ds(s0, n_per_w)], idx_v)

        def start_chunk(ci):
            p = ci % 4
            off = ci * C
            g = pltpu.async_copy(
                table_hbm.at[idx_v.at[pl.ds(off, C)]], bufs[p], gsems[p])
            pp = pltpu.async_copy(
                pos_hbm.at[pl.ds(s0 + off, C)], pbufs[p], psems[p])
            return g, pp

        def add_pos(p):
            buf, pbuf = bufs[p], pbufs[p]
            bpr = D // (2 * _L)

            @plsc.parallel_loop(0, bpr, 1, unroll=2)
            def _(i):
                col = i * 2 * _L
                for r in range(C):
                    w = pbuf[r, pl.ds(i * _L, _L)]
                    a = lax.bitcast_convert_type(
                        lax.shift_left(w, 16), jnp.float32)
                    b = lax.bitcast_convert_type(
                        lax.bitwise_and(w, jnp.int32(-65536)), jnp.float32)
                    plsc.addupdate(buf.at[r, pl.ds(col, _L)], a)
                    plsc.addupdate(buf.at[r, pl.ds(col + _L, _L)], b)

        store_descs = [None, None, None, None]
        in_flight = [start_chunk(0), start_chunk(1), start_chunk(2)]
        for ci in range(n_chunks):
            p = ci % 4
            if ci + 3 < n_chunks:
                nxt = (ci + 3) % 4
                if store_descs[nxt] is not None:
                    store_descs[nxt].wait()
                in_flight.append(start_chunk(ci + 3))
            g, pp = in_flight[ci]
            g.wait()
            pp.wait()
            add_pos(p)
            store_descs[p] = pltpu.async_copy(
                bufs[p], out_hbm.at[pl.ds(wid * n_per_w + ci * C, C)], osems[p])
        for d in store_descs:
            if d is not None:
                d.wait()

    return emb_kernel(x, table, pos).reshape(B, S, D)
